# baseline (speedup 1.0000x reference)
import functools

import jax
import jax.numpy as jnp
from jax import lax
from jax.experimental import pallas as pl
from jax.experimental.pallas import tpu as pltpu
from jax.experimental.pallas import tpu_sc as plsc


def _sc_gather_rows(W_emb, B_emb, x_index):
    T = x_index.shape[0]
    D = W_emb.shape[1]
    mesh = plsc.VectorSubcoreMesh(core_axis_name="c", subcore_axis_name="s")

    @functools.partial(
        pl.kernel,
        mesh=mesh,
        out_type=(
            jax.ShapeDtypeStruct((T, D), jnp.float32),
            jax.ShapeDtypeStruct((T, D), jnp.float32),
        ),
        scratch_types=[
            pltpu.VMEM((T,), jnp.int32),
            pltpu.VMEM((T, D), jnp.float32),
            pltpu.SemaphoreType.DMA,
        ],
        compiler_params=pltpu.CompilerParams(use_tc_tiling_on_sc=False),
    )
    def gather_kernel(w_hbm, b_hbm, idx_hbm, w_out, b_out, idx_v, rows_v, sem):
        cid = lax.axis_index("c")
        sid = lax.axis_index("s")
        wid = sid * 2 + cid

        @pl.when(wid == 0)
        def _():
            pltpu.sync_copy(idx_hbm, idx_v)
            pltpu.async_copy(w_hbm.at[idx_v], rows_v, sem).wait()
            pltpu.sync_copy(rows_v, w_out)

        @pl.when(wid == 1)
        def _():
            pltpu.sync_copy(idx_hbm, idx_v)
            pltpu.async_copy(b_hbm.at[idx_v], rows_v, sem).wait()
            pltpu.sync_copy(rows_v, b_out)

    return gather_kernel(W_emb, B_emb, x_index)


def _tc_body(x_ref, wf_ref, bf_ref, out_ref, *, T, D):
    TD = T * D
    t_ids = lax.broadcasted_iota(jnp.int32, (T, TD), 0)
    j_ids = lax.broadcasted_iota(jnp.int32, (T, TD), 1)
    m = jnp.where(
        (j_ids // D) == t_ids,
        jnp.broadcast_to(wf_ref[...], (T, TD)),
        0.0,
    )
    out_ref[...] = (
        jnp.dot(x_ref[...], m, preferred_element_type=jnp.float32) + bf_ref[...]
    )


def kernel(x, x_index, W_emb, B_emb):
    B, T = x.shape
    D = W_emb.shape[1]
    TD = T * D

    w_rows, b_rows = _sc_gather_rows(W_emb, B_emb, x_index)
    wflat = w_rows.reshape(1, TD)
    bflat = b_rows.reshape(1, TD)

    BBLK = 512
    out2d = pl.pallas_call(
        functools.partial(_tc_body, T=T, D=D),
        grid=(B // BBLK,),
        in_specs=[
            pl.BlockSpec((BBLK, T), lambda i: (i, 0)),
            pl.BlockSpec((1, TD), lambda i: (0, 0)),
            pl.BlockSpec((1, TD), lambda i: (0, 0)),
        ],
        out_specs=pl.BlockSpec((BBLK, TD), lambda i: (i, 0)),
        out_shape=jax.ShapeDtypeStruct((B, TD), jnp.float32),
        compiler_params=pltpu.CompilerParams(
            dimension_semantics=("parallel",),
        ),
    )(x, wflat, bflat)
    return out2d.reshape(B, T, D)
